# baseline (speedup 1.0000x reference)
import jax
import jax.numpy as jnp
from jax.experimental import pallas as pl
from jax.experimental.pallas import tpu as pltpu


_BLOCK_ROWS = 32000


def _colsum_kernel(x_ref, o_ref, acc_ref):
    i = pl.program_id(0)

    @pl.when(i == 0)
    def _init():
        acc_ref[...] = jnp.zeros_like(acc_ref)

    acc_ref[...] += x_ref[...].reshape(-1, 32, 128).sum(axis=0)

    @pl.when(i == pl.num_programs(0) - 1)
    def _fini():
        o_ref[...] = acc_ref[...].sum(axis=0, keepdims=True)


def kernel(x):
    n, c = x.shape
    block = _BLOCK_ROWS
    grid = n // block
    out = pl.pallas_call(
        _colsum_kernel,
        grid=(grid,),
        in_specs=[pl.BlockSpec((block, c), lambda i: (i, 0))],
        out_specs=pl.BlockSpec((1, c), lambda i: (0, 0)),
        out_shape=jax.ShapeDtypeStruct((1, c), x.dtype),
        scratch_shapes=[pltpu.VMEM((32, c), jnp.float32)],
    )(x)
    return out


# speedup vs baseline: 1.0809x; 1.0809x over previous
import jax
import jax.numpy as jnp
from jax.experimental import pallas as pl
from jax.experimental.pallas import tpu as pltpu


_BLOCK_ROWS = 8000


def _colsum_kernel(x1_ref, x2_ref, o_ref, acc_ref):
    i = pl.program_id(0)

    @pl.when(i == 0)
    def _init():
        acc_ref[...] = jnp.zeros_like(acc_ref)

    acc_ref[...] += (
        x1_ref[...].reshape(-1, 32, 128).sum(axis=0)
        + x2_ref[...].reshape(-1, 32, 128).sum(axis=0)
    )

    @pl.when(i == pl.num_programs(0) - 1)
    def _fini():
        o_ref[...] = acc_ref[...].sum(axis=0, keepdims=True)


def kernel(x):
    n, c = x.shape
    block = _BLOCK_ROWS
    grid = (n // 2) // block
    out = pl.pallas_call(
        _colsum_kernel,
        grid=(grid,),
        in_specs=[
            pl.BlockSpec((block, c), lambda i: (i, 0)),
            pl.BlockSpec((block, c), lambda i: (i + grid, 0)),
        ],
        out_specs=pl.BlockSpec((1, c), lambda i: (0, 0)),
        out_shape=jax.ShapeDtypeStruct((1, c), x.dtype),
        scratch_shapes=[pltpu.VMEM((32, c), jnp.float32)],
    )(x, x)
    return out
